# baseline (speedup 1.0000x reference)
import functools

import jax
import jax.numpy as jnp
from jax import lax
from jax.experimental import pallas as pl
from jax.experimental.pallas import tpu as pltpu
from jax.experimental.pallas import tpu_sc as plsc

N = 10000
E = 320000
C = 128
NEG_SLOPE = 0.2

NC = 2
NS = 16
L = 16
NW = NC * NS

K = 112
NCHUNK = 90
EPT = K * NCHUNK
ETOT = NW * EPT
NP = 10240
RPT = NP // NS
KA = 128
CNT0 = 145

BR = 1000


def _alpha_body(x_ref, w_ref, as_ref, ad_ref, asrc_ref, adst_ref,
                gs_ref, gd_ref):
    i = pl.program_id(0)
    u_s = lax.dot_general(w_ref[...], as_ref[...], (((0,), (1,)), ((), ())),
                          preferred_element_type=jnp.float32)
    u_d = lax.dot_general(w_ref[...], ad_ref[...], (((0,), (1,)), ((), ())),
                          preferred_element_type=jnp.float32)
    a_s = lax.dot_general(x_ref[...], u_s, (((1,), (0,)), ((), ())),
                          preferred_element_type=jnp.float32)
    a_d = lax.dot_general(x_ref[...], u_d, (((1,), (0,)), ((), ())),
                          preferred_element_type=jnp.float32)
    asrc_ref[...] = a_s
    adst_ref[...] = a_d

    @pl.when(i == 0)
    def _():
        gs_ref[0, 0] = -jnp.inf
        gd_ref[0, 0] = -jnp.inf

    gs_ref[0, 0] = jnp.maximum(gs_ref[0, 0], jnp.max(a_s))
    gd_ref[0, 0] = jnp.maximum(gd_ref[0, 0], jnp.max(a_d))


def _alphas(x, w, att_s, att_d):
    return pl.pallas_call(
        _alpha_body,
        grid=(N // BR,),
        in_specs=[
            pl.BlockSpec((BR, C), lambda i: (i, 0)),
            pl.BlockSpec((C, C), lambda i: (0, 0)),
            pl.BlockSpec((1, C), lambda i: (0, 0)),
            pl.BlockSpec((1, C), lambda i: (0, 0)),
        ],
        out_specs=[
            pl.BlockSpec((BR, 1), lambda i: (i, 0)),
            pl.BlockSpec((BR, 1), lambda i: (i, 0)),
            pl.BlockSpec((1, 1), lambda i: (0, 0),
                         memory_space=pltpu.MemorySpace.SMEM),
            pl.BlockSpec((1, 1), lambda i: (0, 0),
                         memory_space=pltpu.MemorySpace.SMEM),
        ],
        out_shape=[
            jax.ShapeDtypeStruct((N, 1), jnp.float32),
            jax.ShapeDtypeStruct((N, 1), jnp.float32),
            jax.ShapeDtypeStruct((1, 1), jnp.float32),
            jax.ShapeDtypeStruct((1, 1), jnp.float32),
        ],
    )(x, w, att_s, att_d)


def _proj_body(x_ref, w_ref, xp_ref):
    xp_ref[...] = lax.dot_general(x_ref[...], w_ref[...],
                                  (((1,), (1,)), ((), ())),
                                  preferred_element_type=jnp.float32)


def _project(x, w):
    return pl.pallas_call(
        _proj_body,
        grid=(N // BR,),
        in_specs=[
            pl.BlockSpec((BR, C), lambda i: (i, 0)),
            pl.BlockSpec((C, C), lambda i: (0, 0)),
        ],
        out_specs=pl.BlockSpec((BR, C), lambda i: (i, 0)),
        out_shape=jax.ShapeDtypeStruct((N, C), jnp.float32),
    )(x, w)


_sc_mesh = plsc.VectorSubcoreMesh(core_axis_name="c", subcore_axis_name="s",
                                  num_cores=NC, num_subcores=NS)


@functools.partial(
    pl.kernel,
    out_type=[jax.ShapeDtypeStruct((ETOT,), jnp.float32),
              jax.ShapeDtypeStruct((NW, NP), jnp.float32)],
    mesh=_sc_mesh,
    scratch_types=[
        pltpu.VMEM((N,), jnp.float32),
        pltpu.VMEM((N,), jnp.float32),
        pltpu.VMEM((L,), jnp.float32),
        pltpu.VMEM((EPT,), jnp.int32),
        pltpu.VMEM((EPT,), jnp.int32),
        pltpu.VMEM((EPT,), jnp.float32),
        pltpu.VMEM((NP,), jnp.float32),
        pltpu.VMEM((L,), jnp.int32),
        pltpu.VMEM((L,), jnp.float32),
    ],
    compiler_params=pltpu.CompilerParams(needs_layout_passes=False),
)
def _weights_kernel(asrc_hbm, adst_hbm, g_hbm, srcx_hbm, dstx_hbm,
                    wall_hbm, ws_hbm, asrc_v, adst_v, g_v, si_v, di_v, w_v,
                    sp_v, ksc, csc):
    cid = lax.axis_index("c")
    sid = lax.axis_index("s")
    wid = sid * NC + cid
    ebase = wid * EPT

    pltpu.sync_copy(asrc_hbm, asrc_v)
    pltpu.sync_copy(adst_hbm, adst_v)
    pltpu.sync_copy(g_hbm, g_v)
    pltpu.sync_copy(srcx_hbm.at[pl.ds(ebase, EPT)], si_v)
    pltpu.sync_copy(dstx_hbm.at[pl.ds(ebase, EPT)], di_v)

    def _zero_sp(i, carry):
        sp_v[pl.ds(i * L, L)] = jnp.zeros((L,), jnp.float32)
        return carry

    lax.fori_loop(0, NP // L, _zero_sp, 0)

    gvec = g_v[...]
    lane = lax.broadcasted_iota(jnp.int32, (L,), 0)

    def _group(gi, carry):
        o = gi * L
        sv = si_v[pl.ds(o, L)]
        dv = di_v[pl.ds(o, L)]
        a = plsc.load_gather(asrc_v, [sv]) + plsc.load_gather(adst_v, [dv])
        a = jnp.where(a > 0, a, NEG_SLOPE * a)
        eid = jnp.full((L,), ebase + o, jnp.int32) + lane
        valid = (eid < E) & (sv != dv)
        w = jnp.where(valid, jnp.exp(a - gvec), 0.0)
        w_v[pl.ds(o, L)] = w
        k, v = plsc.sort_key_val(dv, w)
        ksc[...] = k
        cs = plsc.cumsum(v)
        csc[...] = cs
        prev_k = plsc.load_gather(ksc, [jnp.maximum(lane - 1, 0)])
        start = (lane == 0) | (k != prev_k)
        run_start = plsc.cummax(jnp.where(start, lane, 0))
        csb = plsc.load_gather(csc, [jnp.maximum(run_start - 1, 0)])
        csb = jnp.where(run_start == 0, 0.0, csb)
        next_k = plsc.load_gather(ksc, [jnp.minimum(lane + 1, L - 1)])
        end = (lane == L - 1) | (k != next_k)
        plsc.addupdate_scatter(sp_v, [k], cs - csb, mask=end)
        return carry

    lax.fori_loop(0, EPT // L, _group, 0)

    pltpu.sync_copy(w_v, wall_hbm.at[pl.ds(ebase, EPT)])
    pltpu.sync_copy(sp_v, ws_hbm.at[wid])


@functools.partial(
    pl.kernel,
    out_type=[jax.ShapeDtypeStruct((NP, C), jnp.float32),
              jax.ShapeDtypeStruct((NP, C), jnp.float32)],
    mesh=_sc_mesh,
    scratch_types=[
        pltpu.VMEM((K,), jnp.int32),
        pltpu.VMEM((K,), jnp.int32),
        pltpu.VMEM((K,), jnp.int32),
        pltpu.VMEM((K,), jnp.int32),
        pltpu.VMEM((K,), jnp.int32),
        pltpu.VMEM((K,), jnp.int32),
        pltpu.VMEM((K,), jnp.float32),
        pltpu.VMEM((K,), jnp.float32),
        pltpu.VMEM((K,), jnp.float32),
        pltpu.VMEM((K, C), jnp.float32),
        pltpu.VMEM((K, C), jnp.float32),
        pltpu.VMEM((K, C), jnp.float32),
        pltpu.VMEM_SHARED((NP, C), jnp.float32),
        pltpu.SemaphoreType.DMA,
        pltpu.SemaphoreType.DMA,
        pltpu.SemaphoreType.DMA,
    ],
    compiler_params=pltpu.CompilerParams(needs_layout_passes=False),
)
def _rows_kernel(srcx_hbm, dstx_hbm, wall_hbm, xp_hbm, out0_hbm, out1_hbm,
                 si0, si1, si2, di0, di1, di2, w0, w1, w2, b0, b1, b2,
                 acc_sh, sem0, sem1, sem2):
    cid = lax.axis_index("c")
    sid = lax.axis_index("s")
    cnt = jnp.where(cid == 0, CNT0, 2 * NCHUNK - CNT0)
    cbase = sid * (2 * NCHUNK) + cid * CNT0
    ebase = cbase * K
    rbase = sid * RPT
    bufs = ((si0, di0, w0, b0, sem0), (si1, di1, w1, b1, sem1),
            (si2, di2, w2, b2, sem2))

    def _zero_row(e, carry):
        for j in range(C // L):
            b0[e, pl.ds(j * L, L)] = jnp.zeros((L,), jnp.float32)
        return carry

    lax.fori_loop(0, K, _zero_row, 0)
    for r in range(RPT // K):
        pltpu.sync_copy(b0, acc_sh.at[pl.ds(rbase + r * K, K)])
    rrem = RPT - (RPT // K) * K
    pltpu.sync_copy(b0.at[pl.ds(0, rrem)],
                    acc_sh.at[pl.ds(rbase + (RPT // K) * K, rrem)])
    plsc.subcore_barrier()

    for b in range(3):
        si_v, di_v, w_v, b_v, sem = bufs[b]
        pltpu.sync_copy(srcx_hbm.at[pl.ds(ebase + b * K, K)], si_v)
        pltpu.sync_copy(dstx_hbm.at[pl.ds(ebase + b * K, K)], di_v)
        pltpu.sync_copy(wall_hbm.at[pl.ds(ebase + b * K, K)], w_v)
        pltpu.async_copy(xp_hbm.at[si_v], b_v, sem)

    def _triple(i, carry):
        co = i * 3
        for b in range(3):
            si_v, di_v, w_v, b_v, sem = bufs[b]
            ci = co + b

            @pl.when(ci < cnt)
            def _():
                pltpu.make_async_copy(xp_hbm.at[si_v], b_v, sem).wait()

                def _scale_group(gi, carry2):
                    o = gi * L
                    wg = w_v[pl.ds(o, L)]
                    for k in range(L):
                        wb = jnp.full((L,), wg[k], jnp.float32)
                        for j in range(C // L):
                            b_v[o + k, pl.ds(j * L, L)] = (
                                b_v[o + k, pl.ds(j * L, L)] * wb)
                    return carry2

                lax.fori_loop(0, K // L, _scale_group, 0)
                pltpu.sync_copy(b_v, acc_sh.at[di_v], add=True)

                nci = ci + 3

                @pl.when(nci < cnt)
                def _():
                    nb = ebase + nci * K
                    pltpu.sync_copy(srcx_hbm.at[pl.ds(nb, K)], si_v)
                    pltpu.sync_copy(dstx_hbm.at[pl.ds(nb, K)], di_v)
                    pltpu.sync_copy(wall_hbm.at[pl.ds(nb, K)], w_v)
                    pltpu.async_copy(xp_hbm.at[si_v], b_v, sem)

        return carry

    lax.fori_loop(0, (cnt + 2) // 3, _triple, 0)

    plsc.subcore_barrier()

    @pl.when(cid == 0)
    def _():
        for r in range(RPT // KA):
            ro = rbase + r * KA
            pltpu.sync_copy(acc_sh.at[pl.ds(ro, KA)],
                            out0_hbm.at[pl.ds(ro, KA)])

    @pl.when(cid == 1)
    def _():
        for r in range(RPT // KA):
            ro = rbase + r * KA
            pltpu.sync_copy(acc_sh.at[pl.ds(ro, KA)],
                            out1_hbm.at[pl.ds(ro, KA)])


def _norm_body(acc0_ref, acc1_ref, ws_ref, xp_ref, asrc_ref, adst_ref,
               gs_ref, gd_ref, bias_ref, out_ref):
    g = jnp.maximum(gs_ref[0, 0] + gd_ref[0, 0], 0.0)
    a = asrc_ref[...] + adst_ref[...]
    a = jnp.where(a > 0, a, NEG_SLOPE * a)
    w = jnp.exp(a - g)
    s = lax.dot_general(ws_ref[...], jnp.ones((NW, 1), jnp.float32),
                        (((0,), (0,)), ((), ())),
                        preferred_element_type=jnp.float32)
    y = (acc0_ref[...] + acc1_ref[...]) + w * xp_ref[...]
    out_ref[...] = y / (s + w + 1e-16) + bias_ref[...]


def _normalize(acc0, acc1, ws, xp, asrc, adst, gs, gd, bias):
    BN = 1024
    return pl.pallas_call(
        _norm_body,
        grid=(NP // BN,),
        in_specs=[
            pl.BlockSpec((BN, C), lambda i: (i, 0)),
            pl.BlockSpec((BN, C), lambda i: (i, 0)),
            pl.BlockSpec((NW, BN), lambda i: (0, i)),
            pl.BlockSpec((BN, C), lambda i: (i, 0)),
            pl.BlockSpec((BN, 1), lambda i: (i, 0)),
            pl.BlockSpec((BN, 1), lambda i: (i, 0)),
            pl.BlockSpec((1, 1), lambda i: (0, 0),
                         memory_space=pltpu.MemorySpace.SMEM),
            pl.BlockSpec((1, 1), lambda i: (0, 0),
                         memory_space=pltpu.MemorySpace.SMEM),
            pl.BlockSpec((1, C), lambda i: (0, 0)),
        ],
        out_specs=pl.BlockSpec((BN, C), lambda i: (i, 0)),
        out_shape=jax.ShapeDtypeStruct((N, C), jnp.float32),
    )(acc0, acc1, ws, xp, asrc, adst, gs, gd, bias)


def kernel(x, edge_index, W_src, att_src, att_dst, bias):
    att_s = att_src.reshape(1, C)
    att_d = att_dst.reshape(1, C)

    asrc, adst, gs, gd = _alphas(x, W_src, att_s, att_d)

    g = jnp.maximum(gs[0, 0] + gd[0, 0], 0.0)
    g16 = jnp.broadcast_to(g.reshape(1), (L,)).astype(jnp.float32)

    zpad = jnp.zeros((ETOT - E,), jnp.int32)
    srcx = jnp.concatenate([edge_index[0], zpad])
    dstx = jnp.concatenate([edge_index[1], zpad])

    wall, ws = _weights_kernel(asrc.reshape(N), adst.reshape(N), g16,
                               srcx, dstx)
    xp = _project(x, W_src)
    acc0, acc1 = _rows_kernel(srcx, dstx, wall, xp)

    out = _normalize(acc0, acc1, ws, xp, asrc, adst, gs, gd,
                     bias.reshape(1, C))
    return out


# speedup vs baseline: 1.0499x; 1.0499x over previous
import functools

import jax
import jax.numpy as jnp
from jax import lax
from jax.experimental import pallas as pl
from jax.experimental.pallas import tpu as pltpu
from jax.experimental.pallas import tpu_sc as plsc

N = 10000
E = 320000
C = 128
NEG_SLOPE = 0.2

NC = 2
NS = 16
L = 16
NW = NC * NS

K = 112
NCHUNK = 90
EPT = K * NCHUNK
ETOT = NW * EPT
NP = 10240
RPT = NP // NS
KA = 128
CNT0 = 136

BR = 1000


def _alpha_body(x_ref, w_ref, as_ref, ad_ref, asrc_ref, adst_ref,
                gs_ref, gd_ref):
    i = pl.program_id(0)
    u_s = lax.dot_general(w_ref[...], as_ref[...], (((0,), (1,)), ((), ())),
                          preferred_element_type=jnp.float32)
    u_d = lax.dot_general(w_ref[...], ad_ref[...], (((0,), (1,)), ((), ())),
                          preferred_element_type=jnp.float32)
    a_s = lax.dot_general(x_ref[...], u_s, (((1,), (0,)), ((), ())),
                          preferred_element_type=jnp.float32)
    a_d = lax.dot_general(x_ref[...], u_d, (((1,), (0,)), ((), ())),
                          preferred_element_type=jnp.float32)
    asrc_ref[...] = a_s
    adst_ref[...] = a_d

    @pl.when(i == 0)
    def _():
        gs_ref[0, 0] = -jnp.inf
        gd_ref[0, 0] = -jnp.inf

    gs_ref[0, 0] = jnp.maximum(gs_ref[0, 0], jnp.max(a_s))
    gd_ref[0, 0] = jnp.maximum(gd_ref[0, 0], jnp.max(a_d))


def _alphas(x, w, att_s, att_d):
    return pl.pallas_call(
        _alpha_body,
        grid=(N // BR,),
        in_specs=[
            pl.BlockSpec((BR, C), lambda i: (i, 0)),
            pl.BlockSpec((C, C), lambda i: (0, 0)),
            pl.BlockSpec((1, C), lambda i: (0, 0)),
            pl.BlockSpec((1, C), lambda i: (0, 0)),
        ],
        out_specs=[
            pl.BlockSpec((BR, 1), lambda i: (i, 0)),
            pl.BlockSpec((BR, 1), lambda i: (i, 0)),
            pl.BlockSpec((1, 1), lambda i: (0, 0),
                         memory_space=pltpu.MemorySpace.SMEM),
            pl.BlockSpec((1, 1), lambda i: (0, 0),
                         memory_space=pltpu.MemorySpace.SMEM),
        ],
        out_shape=[
            jax.ShapeDtypeStruct((N, 1), jnp.float32),
            jax.ShapeDtypeStruct((N, 1), jnp.float32),
            jax.ShapeDtypeStruct((1, 1), jnp.float32),
            jax.ShapeDtypeStruct((1, 1), jnp.float32),
        ],
    )(x, w, att_s, att_d)


def _proj_body(x_ref, w_ref, xp_ref):
    xp_ref[...] = lax.dot_general(x_ref[...], w_ref[...],
                                  (((1,), (1,)), ((), ())),
                                  preferred_element_type=jnp.float32)


def _project(x, w):
    return pl.pallas_call(
        _proj_body,
        grid=(N // BR,),
        in_specs=[
            pl.BlockSpec((BR, C), lambda i: (i, 0)),
            pl.BlockSpec((C, C), lambda i: (0, 0)),
        ],
        out_specs=pl.BlockSpec((BR, C), lambda i: (i, 0)),
        out_shape=jax.ShapeDtypeStruct((N, C), jnp.float32),
    )(x, w)


_sc_mesh = plsc.VectorSubcoreMesh(core_axis_name="c", subcore_axis_name="s",
                                  num_cores=NC, num_subcores=NS)


@functools.partial(
    pl.kernel,
    out_type=[jax.ShapeDtypeStruct((ETOT,), jnp.float32),
              jax.ShapeDtypeStruct((NW, NP), jnp.float32)],
    mesh=_sc_mesh,
    scratch_types=[
        pltpu.VMEM((N,), jnp.float32),
        pltpu.VMEM((N,), jnp.float32),
        pltpu.VMEM((L,), jnp.float32),
        pltpu.VMEM((EPT,), jnp.int32),
        pltpu.VMEM((EPT,), jnp.int32),
        pltpu.VMEM((EPT,), jnp.float32),
        pltpu.VMEM((NP,), jnp.float32),
        pltpu.VMEM((L,), jnp.int32),
        pltpu.VMEM((L,), jnp.float32),
    ],
    compiler_params=pltpu.CompilerParams(needs_layout_passes=False),
)
def _weights_kernel(asrc_hbm, adst_hbm, g_hbm, srcx_hbm, dstx_hbm,
                    wall_hbm, ws_hbm, asrc_v, adst_v, g_v, si_v, di_v, w_v,
                    sp_v, ksc, csc):
    cid = lax.axis_index("c")
    sid = lax.axis_index("s")
    wid = sid * NC + cid
    ebase = wid * EPT

    pltpu.sync_copy(asrc_hbm, asrc_v)
    pltpu.sync_copy(adst_hbm, adst_v)
    pltpu.sync_copy(g_hbm, g_v)
    pltpu.sync_copy(srcx_hbm.at[pl.ds(ebase, EPT)], si_v)
    pltpu.sync_copy(dstx_hbm.at[pl.ds(ebase, EPT)], di_v)

    def _zero_sp(i, carry):
        sp_v[pl.ds(i * L, L)] = jnp.zeros((L,), jnp.float32)
        return carry

    lax.fori_loop(0, NP // L, _zero_sp, 0)

    gvec = g_v[...]
    lane = lax.broadcasted_iota(jnp.int32, (L,), 0)

    def _group(gi, carry):
        o = gi * L
        sv = si_v[pl.ds(o, L)]
        dv = di_v[pl.ds(o, L)]
        a = plsc.load_gather(asrc_v, [sv]) + plsc.load_gather(adst_v, [dv])
        a = jnp.where(a > 0, a, NEG_SLOPE * a)
        eid = jnp.full((L,), ebase + o, jnp.int32) + lane
        valid = (eid < E) & (sv != dv)
        w = jnp.where(valid, jnp.exp(a - gvec), 0.0)
        w_v[pl.ds(o, L)] = w
        k, v = plsc.sort_key_val(dv, w)
        ksc[...] = k
        cs = plsc.cumsum(v)
        csc[...] = cs
        prev_k = plsc.load_gather(ksc, [jnp.maximum(lane - 1, 0)])
        start = (lane == 0) | (k != prev_k)
        run_start = plsc.cummax(jnp.where(start, lane, 0))
        csb = plsc.load_gather(csc, [jnp.maximum(run_start - 1, 0)])
        csb = jnp.where(run_start == 0, 0.0, csb)
        next_k = plsc.load_gather(ksc, [jnp.minimum(lane + 1, L - 1)])
        end = (lane == L - 1) | (k != next_k)
        plsc.addupdate_scatter(sp_v, [k], cs - csb, mask=end)
        return carry

    lax.fori_loop(0, EPT // L, _group, 0)

    pltpu.sync_copy(w_v, wall_hbm.at[pl.ds(ebase, EPT)])
    pltpu.sync_copy(sp_v, ws_hbm.at[wid])


@functools.partial(
    pl.kernel,
    out_type=[jax.ShapeDtypeStruct((NP, C), jnp.float32),
              jax.ShapeDtypeStruct((NP, C), jnp.float32)],
    mesh=_sc_mesh,
    scratch_types=[
        pltpu.VMEM((K,), jnp.int32),
        pltpu.VMEM((K,), jnp.int32),
        pltpu.VMEM((K,), jnp.int32),
        pltpu.VMEM((K,), jnp.int32),
        pltpu.VMEM((K,), jnp.int32),
        pltpu.VMEM((K,), jnp.int32),
        pltpu.VMEM((K,), jnp.float32),
        pltpu.VMEM((K,), jnp.float32),
        pltpu.VMEM((K,), jnp.float32),
        pltpu.VMEM((K, C), jnp.float32),
        pltpu.VMEM((K, C), jnp.float32),
        pltpu.VMEM((K, C), jnp.float32),
        pltpu.VMEM_SHARED((NP, C), jnp.float32),
        pltpu.SemaphoreType.DMA,
        pltpu.SemaphoreType.DMA,
        pltpu.SemaphoreType.DMA,
    ],
    compiler_params=pltpu.CompilerParams(needs_layout_passes=False),
)
def _rows_kernel(srcx_hbm, dstx_hbm, wall_hbm, xp_hbm, out0_hbm, out1_hbm,
                 si0, si1, si2, di0, di1, di2, w0, w1, w2, b0, b1, b2,
                 acc_sh, sem0, sem1, sem2):
    cid = lax.axis_index("c")
    sid = lax.axis_index("s")
    cnt = jnp.where(cid == 0, CNT0, 2 * NCHUNK - CNT0)
    cbase = sid * (2 * NCHUNK) + cid * CNT0
    ebase = cbase * K
    rbase = sid * RPT
    bufs = ((si0, di0, w0, b0, sem0), (si1, di1, w1, b1, sem1),
            (si2, di2, w2, b2, sem2))

    def _zero_row(e, carry):
        for j in range(C // L):
            b0[e, pl.ds(j * L, L)] = jnp.zeros((L,), jnp.float32)
        return carry

    lax.fori_loop(0, K, _zero_row, 0)
    for r in range(RPT // K):
        pltpu.sync_copy(b0, acc_sh.at[pl.ds(rbase + r * K, K)])
    rrem = RPT - (RPT // K) * K
    pltpu.sync_copy(b0.at[pl.ds(0, rrem)],
                    acc_sh.at[pl.ds(rbase + (RPT // K) * K, rrem)])
    plsc.subcore_barrier()

    for b in range(3):
        si_v, di_v, w_v, b_v, sem = bufs[b]
        pltpu.sync_copy(srcx_hbm.at[pl.ds(ebase + b * K, K)], si_v)
        pltpu.sync_copy(dstx_hbm.at[pl.ds(ebase + b * K, K)], di_v)
        pltpu.sync_copy(wall_hbm.at[pl.ds(ebase + b * K, K)], w_v)
        pltpu.async_copy(xp_hbm.at[si_v], b_v, sem)

    def _triple(i, carry):
        co = i * 3
        for b in range(3):
            si_v, di_v, w_v, b_v, sem = bufs[b]
            ci = co + b

            @pl.when(ci < cnt)
            def _():
                pltpu.make_async_copy(xp_hbm.at[si_v], b_v, sem).wait()

                def _scale_group(gi, carry2):
                    o = gi * L
                    wg = w_v[pl.ds(o, L)]
                    for k in range(L):
                        wb = jnp.full((L,), wg[k], jnp.float32)
                        for j in range(C // L):
                            b_v[o + k, pl.ds(j * L, L)] = (
                                b_v[o + k, pl.ds(j * L, L)] * wb)
                    return carry2

                lax.fori_loop(0, K // L, _scale_group, 0)
                pltpu.sync_copy(b_v, acc_sh.at[di_v], add=True)

                nci = ci + 3

                @pl.when(nci < cnt)
                def _():
                    nb = ebase + nci * K
                    pltpu.sync_copy(srcx_hbm.at[pl.ds(nb, K)], si_v)
                    pltpu.sync_copy(dstx_hbm.at[pl.ds(nb, K)], di_v)
                    pltpu.sync_copy(wall_hbm.at[pl.ds(nb, K)], w_v)
                    pltpu.async_copy(xp_hbm.at[si_v], b_v, sem)

        return carry

    lax.fori_loop(0, (cnt + 2) // 3, _triple, 0)

    plsc.subcore_barrier()

    @pl.when(cid == 0)
    def _():
        for r in range(RPT // KA):
            ro = rbase + r * KA
            pltpu.sync_copy(acc_sh.at[pl.ds(ro, KA)],
                            out0_hbm.at[pl.ds(ro, KA)])

    @pl.when(cid == 1)
    def _():
        for r in range(RPT // KA):
            ro = rbase + r * KA
            pltpu.sync_copy(acc_sh.at[pl.ds(ro, KA)],
                            out1_hbm.at[pl.ds(ro, KA)])


def _norm_body(acc0_ref, acc1_ref, ws_ref, xp_ref, asrc_ref, adst_ref,
               gs_ref, gd_ref, bias_ref, out_ref):
    g = jnp.maximum(gs_ref[0, 0] + gd_ref[0, 0], 0.0)
    a = asrc_ref[...] + adst_ref[...]
    a = jnp.where(a > 0, a, NEG_SLOPE * a)
    w = jnp.exp(a - g)
    s = lax.dot_general(ws_ref[...], jnp.ones((NW, 1), jnp.float32),
                        (((0,), (0,)), ((), ())),
                        preferred_element_type=jnp.float32)
    y = (acc0_ref[...] + acc1_ref[...]) + w * xp_ref[...]
    out_ref[...] = y / (s + w + 1e-16) + bias_ref[...]


def _normalize(acc0, acc1, ws, xp, asrc, adst, gs, gd, bias):
    BN = 1024
    return pl.pallas_call(
        _norm_body,
        grid=(NP // BN,),
        in_specs=[
            pl.BlockSpec((BN, C), lambda i: (i, 0)),
            pl.BlockSpec((BN, C), lambda i: (i, 0)),
            pl.BlockSpec((NW, BN), lambda i: (0, i)),
            pl.BlockSpec((BN, C), lambda i: (i, 0)),
            pl.BlockSpec((BN, 1), lambda i: (i, 0)),
            pl.BlockSpec((BN, 1), lambda i: (i, 0)),
            pl.BlockSpec((1, 1), lambda i: (0, 0),
                         memory_space=pltpu.MemorySpace.SMEM),
            pl.BlockSpec((1, 1), lambda i: (0, 0),
                         memory_space=pltpu.MemorySpace.SMEM),
            pl.BlockSpec((1, C), lambda i: (0, 0)),
        ],
        out_specs=pl.BlockSpec((BN, C), lambda i: (i, 0)),
        out_shape=jax.ShapeDtypeStruct((N, C), jnp.float32),
    )(acc0, acc1, ws, xp, asrc, adst, gs, gd, bias)


def kernel(x, edge_index, W_src, att_src, att_dst, bias):
    att_s = att_src.reshape(1, C)
    att_d = att_dst.reshape(1, C)

    asrc, adst, gs, gd = _alphas(x, W_src, att_s, att_d)

    g = jnp.maximum(gs[0, 0] + gd[0, 0], 0.0)
    g16 = jnp.broadcast_to(g.reshape(1), (L,)).astype(jnp.float32)

    zpad = jnp.zeros((ETOT - E,), jnp.int32)
    srcx = jnp.concatenate([edge_index[0], zpad])
    dstx = jnp.concatenate([edge_index[1], zpad])

    wall, ws = _weights_kernel(asrc.reshape(N), adst.reshape(N), g16,
                               srcx, dstx)
    xp = _project(x, W_src)
    acc0, acc1 = _rows_kernel(srcx, dstx, wall, xp)

    out = _normalize(acc0, acc1, ws, xp, asrc, adst, gs, gd,
                     bias.reshape(1, C))
    return out


# speedup vs baseline: 1.1441x; 1.0898x over previous
import functools

import jax
import jax.numpy as jnp
from jax import lax
from jax.experimental import pallas as pl
from jax.experimental.pallas import tpu as pltpu
from jax.experimental.pallas import tpu_sc as plsc

N = 10000
E = 320000
C = 128
NEG_SLOPE = 0.2

NC = 2
NS = 16
L = 16
NW = NC * NS

K = 112
NCHUNK = 90
EPT = K * NCHUNK
ETOT = NW * EPT
NP = 10240
RPT = NP // NS
KA = 128
CNT0 = 120

BR = 1000


def _alpha_body(x_ref, w_ref, as_ref, ad_ref, asrc_ref, adst_ref,
                gs_ref, gd_ref):
    i = pl.program_id(0)
    u_s = lax.dot_general(w_ref[...], as_ref[...], (((0,), (1,)), ((), ())),
                          preferred_element_type=jnp.float32)
    u_d = lax.dot_general(w_ref[...], ad_ref[...], (((0,), (1,)), ((), ())),
                          preferred_element_type=jnp.float32)
    a_s = lax.dot_general(x_ref[...], u_s, (((1,), (0,)), ((), ())),
                          preferred_element_type=jnp.float32)
    a_d = lax.dot_general(x_ref[...], u_d, (((1,), (0,)), ((), ())),
                          preferred_element_type=jnp.float32)
    asrc_ref[...] = a_s
    adst_ref[...] = a_d

    @pl.when(i == 0)
    def _():
        gs_ref[0, 0] = -jnp.inf
        gd_ref[0, 0] = -jnp.inf

    gs_ref[0, 0] = jnp.maximum(gs_ref[0, 0], jnp.max(a_s))
    gd_ref[0, 0] = jnp.maximum(gd_ref[0, 0], jnp.max(a_d))


def _alphas(x, w, att_s, att_d):
    return pl.pallas_call(
        _alpha_body,
        grid=(N // BR,),
        in_specs=[
            pl.BlockSpec((BR, C), lambda i: (i, 0)),
            pl.BlockSpec((C, C), lambda i: (0, 0)),
            pl.BlockSpec((1, C), lambda i: (0, 0)),
            pl.BlockSpec((1, C), lambda i: (0, 0)),
        ],
        out_specs=[
            pl.BlockSpec((BR, 1), lambda i: (i, 0)),
            pl.BlockSpec((BR, 1), lambda i: (i, 0)),
            pl.BlockSpec((1, 1), lambda i: (0, 0),
                         memory_space=pltpu.MemorySpace.SMEM),
            pl.BlockSpec((1, 1), lambda i: (0, 0),
                         memory_space=pltpu.MemorySpace.SMEM),
        ],
        out_shape=[
            jax.ShapeDtypeStruct((N, 1), jnp.float32),
            jax.ShapeDtypeStruct((N, 1), jnp.float32),
            jax.ShapeDtypeStruct((1, 1), jnp.float32),
            jax.ShapeDtypeStruct((1, 1), jnp.float32),
        ],
    )(x, w, att_s, att_d)


def _proj_body(x_ref, w_ref, xp_ref):
    xp_ref[...] = lax.dot_general(x_ref[...], w_ref[...],
                                  (((1,), (1,)), ((), ())),
                                  preferred_element_type=jnp.float32)


def _project(x, w):
    return pl.pallas_call(
        _proj_body,
        grid=(N // BR,),
        in_specs=[
            pl.BlockSpec((BR, C), lambda i: (i, 0)),
            pl.BlockSpec((C, C), lambda i: (0, 0)),
        ],
        out_specs=pl.BlockSpec((BR, C), lambda i: (i, 0)),
        out_shape=jax.ShapeDtypeStruct((N, C), jnp.float32),
    )(x, w)


_sc_mesh = plsc.VectorSubcoreMesh(core_axis_name="c", subcore_axis_name="s",
                                  num_cores=NC, num_subcores=NS)


@functools.partial(
    pl.kernel,
    out_type=[jax.ShapeDtypeStruct((ETOT,), jnp.float32),
              jax.ShapeDtypeStruct((NW, NP), jnp.float32)],
    mesh=_sc_mesh,
    scratch_types=[
        pltpu.VMEM((N,), jnp.float32),
        pltpu.VMEM((N,), jnp.float32),
        pltpu.VMEM((L,), jnp.float32),
        pltpu.VMEM((EPT,), jnp.int32),
        pltpu.VMEM((EPT,), jnp.int32),
        pltpu.VMEM((EPT,), jnp.float32),
        pltpu.VMEM((NP,), jnp.float32),
        pltpu.VMEM((L,), jnp.int32),
        pltpu.VMEM((L,), jnp.float32),
    ],
    compiler_params=pltpu.CompilerParams(needs_layout_passes=False),
)
def _weights_kernel(asrc_hbm, adst_hbm, g_hbm, srcx_hbm, dstx_hbm,
                    wall_hbm, ws_hbm, asrc_v, adst_v, g_v, si_v, di_v, w_v,
                    sp_v, ksc, csc):
    cid = lax.axis_index("c")
    sid = lax.axis_index("s")
    wid = sid * NC + cid
    ebase = wid * EPT

    pltpu.sync_copy(asrc_hbm, asrc_v)
    pltpu.sync_copy(adst_hbm, adst_v)
    pltpu.sync_copy(g_hbm, g_v)
    pltpu.sync_copy(srcx_hbm.at[pl.ds(ebase, EPT)], si_v)
    pltpu.sync_copy(dstx_hbm.at[pl.ds(ebase, EPT)], di_v)

    def _zero_sp(i, carry):
        sp_v[pl.ds(i * L, L)] = jnp.zeros((L,), jnp.float32)
        return carry

    lax.fori_loop(0, NP // L, _zero_sp, 0)

    gvec = g_v[...]
    lane = lax.broadcasted_iota(jnp.int32, (L,), 0)

    def _group(gi, carry):
        o = gi * L
        sv = si_v[pl.ds(o, L)]
        dv = di_v[pl.ds(o, L)]
        a = plsc.load_gather(asrc_v, [sv]) + plsc.load_gather(adst_v, [dv])
        a = jnp.where(a > 0, a, NEG_SLOPE * a)
        eid = jnp.full((L,), ebase + o, jnp.int32) + lane
        valid = (eid < E) & (sv != dv)
        w = jnp.where(valid, jnp.exp(a - gvec), 0.0)
        w_v[pl.ds(o, L)] = w
        k, v = plsc.sort_key_val(dv, w)
        ksc[...] = k
        cs = plsc.cumsum(v)
        csc[...] = cs
        prev_k = plsc.load_gather(ksc, [jnp.maximum(lane - 1, 0)])
        start = (lane == 0) | (k != prev_k)
        run_start = plsc.cummax(jnp.where(start, lane, 0))
        csb = plsc.load_gather(csc, [jnp.maximum(run_start - 1, 0)])
        csb = jnp.where(run_start == 0, 0.0, csb)
        next_k = plsc.load_gather(ksc, [jnp.minimum(lane + 1, L - 1)])
        end = (lane == L - 1) | (k != next_k)
        plsc.addupdate_scatter(sp_v, [k], cs - csb, mask=end)
        return carry

    lax.fori_loop(0, EPT // L, _group, 0)

    pltpu.sync_copy(w_v, wall_hbm.at[pl.ds(ebase, EPT)])
    pltpu.sync_copy(sp_v, ws_hbm.at[wid])


@functools.partial(
    pl.kernel,
    out_type=[jax.ShapeDtypeStruct((NP, C), jnp.float32),
              jax.ShapeDtypeStruct((NP, C), jnp.float32)],
    mesh=_sc_mesh,
    scratch_types=[
        pltpu.VMEM((K,), jnp.int32),
        pltpu.VMEM((K,), jnp.int32),
        pltpu.VMEM((K,), jnp.int32),
        pltpu.VMEM((K,), jnp.int32),
        pltpu.VMEM((K,), jnp.int32),
        pltpu.VMEM((K,), jnp.int32),
        pltpu.VMEM((K,), jnp.float32),
        pltpu.VMEM((K,), jnp.float32),
        pltpu.VMEM((K,), jnp.float32),
        pltpu.VMEM((K, C), jnp.float32),
        pltpu.VMEM((K, C), jnp.float32),
        pltpu.VMEM((K, C), jnp.float32),
        pltpu.VMEM_SHARED((NP, C), jnp.float32),
        pltpu.SemaphoreType.DMA,
        pltpu.SemaphoreType.DMA,
        pltpu.SemaphoreType.DMA,
    ],
    compiler_params=pltpu.CompilerParams(needs_layout_passes=False),
)
def _rows_kernel(srcx_hbm, dstx_hbm, wall_hbm, xp_hbm, out0_hbm, out1_hbm,
                 si0, si1, si2, di0, di1, di2, w0, w1, w2, b0, b1, b2,
                 acc_sh, sem0, sem1, sem2):
    cid = lax.axis_index("c")
    sid = lax.axis_index("s")
    cnt = jnp.where(cid == 0, CNT0, 2 * NCHUNK - CNT0)
    cbase = sid * (2 * NCHUNK) + cid * CNT0
    ebase = cbase * K
    rbase = sid * RPT
    bufs = ((si0, di0, w0, b0, sem0), (si1, di1, w1, b1, sem1),
            (si2, di2, w2, b2, sem2))

    def _zero_row(e, carry):
        for j in range(C // L):
            b0[e, pl.ds(j * L, L)] = jnp.zeros((L,), jnp.float32)
        return carry

    lax.fori_loop(0, K, _zero_row, 0)
    for r in range(RPT // K):
        pltpu.sync_copy(b0, acc_sh.at[pl.ds(rbase + r * K, K)])
    rrem = RPT - (RPT // K) * K
    pltpu.sync_copy(b0.at[pl.ds(0, rrem)],
                    acc_sh.at[pl.ds(rbase + (RPT // K) * K, rrem)])
    plsc.subcore_barrier()

    for b in range(3):
        si_v, di_v, w_v, b_v, sem = bufs[b]
        pltpu.sync_copy(srcx_hbm.at[pl.ds(ebase + b * K, K)], si_v)
        pltpu.sync_copy(dstx_hbm.at[pl.ds(ebase + b * K, K)], di_v)
        pltpu.sync_copy(wall_hbm.at[pl.ds(ebase + b * K, K)], w_v)
        pltpu.async_copy(xp_hbm.at[si_v], b_v, sem)

    def _triple(i, carry):
        co = i * 3
        for b in range(3):
            si_v, di_v, w_v, b_v, sem = bufs[b]
            ci = co + b

            @pl.when(ci < cnt)
            def _():
                pltpu.make_async_copy(xp_hbm.at[si_v], b_v, sem).wait()

                def _scale_group(gi, carry2):
                    o = gi * L
                    wg = w_v[pl.ds(o, L)]
                    for k in range(L):
                        wb = jnp.full((L,), wg[k], jnp.float32)
                        for j in range(C // L):
                            b_v[o + k, pl.ds(j * L, L)] = (
                                b_v[o + k, pl.ds(j * L, L)] * wb)
                    return carry2

                lax.fori_loop(0, K // L, _scale_group, 0)
                pltpu.sync_copy(b_v, acc_sh.at[di_v], add=True)

                nci = ci + 3

                @pl.when(nci < cnt)
                def _():
                    nb = ebase + nci * K
                    pltpu.sync_copy(srcx_hbm.at[pl.ds(nb, K)], si_v)
                    pltpu.sync_copy(dstx_hbm.at[pl.ds(nb, K)], di_v)
                    pltpu.sync_copy(wall_hbm.at[pl.ds(nb, K)], w_v)
                    pltpu.async_copy(xp_hbm.at[si_v], b_v, sem)

        return carry

    lax.fori_loop(0, (cnt + 2) // 3, _triple, 0)

    plsc.subcore_barrier()

    @pl.when(cid == 0)
    def _():
        for r in range(RPT // KA):
            ro = rbase + r * KA
            pltpu.sync_copy(acc_sh.at[pl.ds(ro, KA)],
                            out0_hbm.at[pl.ds(ro, KA)])

    @pl.when(cid == 1)
    def _():
        for r in range(RPT // KA):
            ro = rbase + r * KA
            pltpu.sync_copy(acc_sh.at[pl.ds(ro, KA)],
                            out1_hbm.at[pl.ds(ro, KA)])


def _norm_body(acc0_ref, acc1_ref, ws_ref, xp_ref, asrc_ref, adst_ref,
               gs_ref, gd_ref, bias_ref, out_ref):
    g = jnp.maximum(gs_ref[0, 0] + gd_ref[0, 0], 0.0)
    a = asrc_ref[...] + adst_ref[...]
    a = jnp.where(a > 0, a, NEG_SLOPE * a)
    w = jnp.exp(a - g)
    s = lax.dot_general(ws_ref[...], jnp.ones((NW, 1), jnp.float32),
                        (((0,), (0,)), ((), ())),
                        preferred_element_type=jnp.float32)
    y = (acc0_ref[...] + acc1_ref[...]) + w * xp_ref[...]
    out_ref[...] = y / (s + w + 1e-16) + bias_ref[...]


def _normalize(acc0, acc1, ws, xp, asrc, adst, gs, gd, bias):
    BN = 1024
    return pl.pallas_call(
        _norm_body,
        grid=(NP // BN,),
        in_specs=[
            pl.BlockSpec((BN, C), lambda i: (i, 0)),
            pl.BlockSpec((BN, C), lambda i: (i, 0)),
            pl.BlockSpec((NW, BN), lambda i: (0, i)),
            pl.BlockSpec((BN, C), lambda i: (i, 0)),
            pl.BlockSpec((BN, 1), lambda i: (i, 0)),
            pl.BlockSpec((BN, 1), lambda i: (i, 0)),
            pl.BlockSpec((1, 1), lambda i: (0, 0),
                         memory_space=pltpu.MemorySpace.SMEM),
            pl.BlockSpec((1, 1), lambda i: (0, 0),
                         memory_space=pltpu.MemorySpace.SMEM),
            pl.BlockSpec((1, C), lambda i: (0, 0)),
        ],
        out_specs=pl.BlockSpec((BN, C), lambda i: (i, 0)),
        out_shape=jax.ShapeDtypeStruct((N, C), jnp.float32),
    )(acc0, acc1, ws, xp, asrc, adst, gs, gd, bias)


def kernel(x, edge_index, W_src, att_src, att_dst, bias):
    att_s = att_src.reshape(1, C)
    att_d = att_dst.reshape(1, C)

    asrc, adst, gs, gd = _alphas(x, W_src, att_s, att_d)

    g = jnp.maximum(gs[0, 0] + gd[0, 0], 0.0)
    g16 = jnp.broadcast_to(g.reshape(1), (L,)).astype(jnp.float32)

    zpad = jnp.zeros((ETOT - E,), jnp.int32)
    srcx = jnp.concatenate([edge_index[0], zpad])
    dstx = jnp.concatenate([edge_index[1], zpad])

    wall, ws = _weights_kernel(asrc.reshape(N), adst.reshape(N), g16,
                               srcx, dstx)
    xp = _project(x, W_src)
    acc0, acc1 = _rows_kernel(srcx, dstx, wall, xp)

    out = _normalize(acc0, acc1, ws, xp, asrc, adst, gs, gd,
                     bias.reshape(1, C))
    return out


# speedup vs baseline: 1.1969x; 1.0461x over previous
import functools

import jax
import jax.numpy as jnp
from jax import lax
from jax.experimental import pallas as pl
from jax.experimental.pallas import tpu as pltpu
from jax.experimental.pallas import tpu_sc as plsc

N = 10000
E = 320000
C = 128
NEG_SLOPE = 0.2

NC = 2
NS = 16
L = 16
NW = NC * NS

K = 112
NCHUNK = 90
EPT = K * NCHUNK
ETOT = NW * EPT
NP = 10240
RPT = NP // NS
KA = 128
CNT0 = 112

BR = 1000


def _alpha_body(x_ref, w_ref, as_ref, ad_ref, asrc_ref, adst_ref,
                gs_ref, gd_ref):
    i = pl.program_id(0)
    u_s = lax.dot_general(w_ref[...], as_ref[...], (((0,), (1,)), ((), ())),
                          preferred_element_type=jnp.float32)
    u_d = lax.dot_general(w_ref[...], ad_ref[...], (((0,), (1,)), ((), ())),
                          preferred_element_type=jnp.float32)
    a_s = lax.dot_general(x_ref[...], u_s, (((1,), (0,)), ((), ())),
                          preferred_element_type=jnp.float32)
    a_d = lax.dot_general(x_ref[...], u_d, (((1,), (0,)), ((), ())),
                          preferred_element_type=jnp.float32)
    asrc_ref[...] = a_s
    adst_ref[...] = a_d

    @pl.when(i == 0)
    def _():
        gs_ref[0, 0] = -jnp.inf
        gd_ref[0, 0] = -jnp.inf

    gs_ref[0, 0] = jnp.maximum(gs_ref[0, 0], jnp.max(a_s))
    gd_ref[0, 0] = jnp.maximum(gd_ref[0, 0], jnp.max(a_d))


def _alphas(x, w, att_s, att_d):
    return pl.pallas_call(
        _alpha_body,
        grid=(N // BR,),
        in_specs=[
            pl.BlockSpec((BR, C), lambda i: (i, 0)),
            pl.BlockSpec((C, C), lambda i: (0, 0)),
            pl.BlockSpec((1, C), lambda i: (0, 0)),
            pl.BlockSpec((1, C), lambda i: (0, 0)),
        ],
        out_specs=[
            pl.BlockSpec((BR, 1), lambda i: (i, 0)),
            pl.BlockSpec((BR, 1), lambda i: (i, 0)),
            pl.BlockSpec((1, 1), lambda i: (0, 0),
                         memory_space=pltpu.MemorySpace.SMEM),
            pl.BlockSpec((1, 1), lambda i: (0, 0),
                         memory_space=pltpu.MemorySpace.SMEM),
        ],
        out_shape=[
            jax.ShapeDtypeStruct((N, 1), jnp.float32),
            jax.ShapeDtypeStruct((N, 1), jnp.float32),
            jax.ShapeDtypeStruct((1, 1), jnp.float32),
            jax.ShapeDtypeStruct((1, 1), jnp.float32),
        ],
    )(x, w, att_s, att_d)


def _proj_body(x_ref, w_ref, xp_ref):
    xp_ref[...] = lax.dot_general(x_ref[...], w_ref[...],
                                  (((1,), (1,)), ((), ())),
                                  preferred_element_type=jnp.float32)


def _project(x, w):
    return pl.pallas_call(
        _proj_body,
        grid=(N // BR,),
        in_specs=[
            pl.BlockSpec((BR, C), lambda i: (i, 0)),
            pl.BlockSpec((C, C), lambda i: (0, 0)),
        ],
        out_specs=pl.BlockSpec((BR, C), lambda i: (i, 0)),
        out_shape=jax.ShapeDtypeStruct((N, C), jnp.float32),
    )(x, w)


_sc_mesh = plsc.VectorSubcoreMesh(core_axis_name="c", subcore_axis_name="s",
                                  num_cores=NC, num_subcores=NS)


@functools.partial(
    pl.kernel,
    out_type=[jax.ShapeDtypeStruct((ETOT,), jnp.float32),
              jax.ShapeDtypeStruct((NW, NP), jnp.float32)],
    mesh=_sc_mesh,
    scratch_types=[
        pltpu.VMEM((N,), jnp.float32),
        pltpu.VMEM((N,), jnp.float32),
        pltpu.VMEM((L,), jnp.float32),
        pltpu.VMEM((EPT,), jnp.int32),
        pltpu.VMEM((EPT,), jnp.int32),
        pltpu.VMEM((EPT,), jnp.float32),
        pltpu.VMEM((NP,), jnp.float32),
        pltpu.VMEM((L,), jnp.int32),
        pltpu.VMEM((L,), jnp.float32),
    ],
    compiler_params=pltpu.CompilerParams(needs_layout_passes=False),
)
def _weights_kernel(asrc_hbm, adst_hbm, g_hbm, srcx_hbm, dstx_hbm,
                    wall_hbm, ws_hbm, asrc_v, adst_v, g_v, si_v, di_v, w_v,
                    sp_v, ksc, csc):
    cid = lax.axis_index("c")
    sid = lax.axis_index("s")
    wid = sid * NC + cid
    ebase = wid * EPT

    pltpu.sync_copy(asrc_hbm, asrc_v)
    pltpu.sync_copy(adst_hbm, adst_v)
    pltpu.sync_copy(g_hbm, g_v)
    pltpu.sync_copy(srcx_hbm.at[pl.ds(ebase, EPT)], si_v)
    pltpu.sync_copy(dstx_hbm.at[pl.ds(ebase, EPT)], di_v)

    def _zero_sp(i, carry):
        sp_v[pl.ds(i * L, L)] = jnp.zeros((L,), jnp.float32)
        return carry

    lax.fori_loop(0, NP // L, _zero_sp, 0)

    gvec = g_v[...]
    lane = lax.broadcasted_iota(jnp.int32, (L,), 0)

    def _group(gi, carry):
        o = gi * L
        sv = si_v[pl.ds(o, L)]
        dv = di_v[pl.ds(o, L)]
        a = plsc.load_gather(asrc_v, [sv]) + plsc.load_gather(adst_v, [dv])
        a = jnp.where(a > 0, a, NEG_SLOPE * a)
        eid = jnp.full((L,), ebase + o, jnp.int32) + lane
        valid = (eid < E) & (sv != dv)
        w = jnp.where(valid, jnp.exp(a - gvec), 0.0)
        w_v[pl.ds(o, L)] = w
        k, v = plsc.sort_key_val(dv, w)
        ksc[...] = k
        cs = plsc.cumsum(v)
        csc[...] = cs
        prev_k = plsc.load_gather(ksc, [jnp.maximum(lane - 1, 0)])
        start = (lane == 0) | (k != prev_k)
        run_start = plsc.cummax(jnp.where(start, lane, 0))
        csb = plsc.load_gather(csc, [jnp.maximum(run_start - 1, 0)])
        csb = jnp.where(run_start == 0, 0.0, csb)
        next_k = plsc.load_gather(ksc, [jnp.minimum(lane + 1, L - 1)])
        end = (lane == L - 1) | (k != next_k)
        plsc.addupdate_scatter(sp_v, [k], cs - csb, mask=end)
        return carry

    lax.fori_loop(0, EPT // L, _group, 0)

    pltpu.sync_copy(w_v, wall_hbm.at[pl.ds(ebase, EPT)])
    pltpu.sync_copy(sp_v, ws_hbm.at[wid])


@functools.partial(
    pl.kernel,
    out_type=[jax.ShapeDtypeStruct((NP, C), jnp.float32),
              jax.ShapeDtypeStruct((NP, C), jnp.float32)],
    mesh=_sc_mesh,
    scratch_types=[
        pltpu.VMEM((K,), jnp.int32),
        pltpu.VMEM((K,), jnp.int32),
        pltpu.VMEM((K,), jnp.int32),
        pltpu.VMEM((K,), jnp.int32),
        pltpu.VMEM((K,), jnp.int32),
        pltpu.VMEM((K,), jnp.int32),
        pltpu.VMEM((K,), jnp.float32),
        pltpu.VMEM((K,), jnp.float32),
        pltpu.VMEM((K,), jnp.float32),
        pltpu.VMEM((K, C), jnp.float32),
        pltpu.VMEM((K, C), jnp.float32),
        pltpu.VMEM((K, C), jnp.float32),
        pltpu.VMEM_SHARED((NP, C), jnp.float32),
        pltpu.SemaphoreType.DMA,
        pltpu.SemaphoreType.DMA,
        pltpu.SemaphoreType.DMA,
    ],
    compiler_params=pltpu.CompilerParams(needs_layout_passes=False),
)
def _rows_kernel(srcx_hbm, dstx_hbm, wall_hbm, xp_hbm, out0_hbm, out1_hbm,
                 si0, si1, si2, di0, di1, di2, w0, w1, w2, b0, b1, b2,
                 acc_sh, sem0, sem1, sem2):
    cid = lax.axis_index("c")
    sid = lax.axis_index("s")
    cnt = jnp.where(cid == 0, CNT0, 2 * NCHUNK - CNT0)
    cbase = sid * (2 * NCHUNK) + cid * CNT0
    ebase = cbase * K
    rbase = sid * RPT
    bufs = ((si0, di0, w0, b0, sem0), (si1, di1, w1, b1, sem1),
            (si2, di2, w2, b2, sem2))

    def _zero_row(e, carry):
        for j in range(C // L):
            b0[e, pl.ds(j * L, L)] = jnp.zeros((L,), jnp.float32)
        return carry

    lax.fori_loop(0, K, _zero_row, 0)
    for r in range(RPT // K):
        pltpu.sync_copy(b0, acc_sh.at[pl.ds(rbase + r * K, K)])
    rrem = RPT - (RPT // K) * K
    pltpu.sync_copy(b0.at[pl.ds(0, rrem)],
                    acc_sh.at[pl.ds(rbase + (RPT // K) * K, rrem)])
    plsc.subcore_barrier()

    for b in range(3):
        si_v, di_v, w_v, b_v, sem = bufs[b]
        pltpu.sync_copy(srcx_hbm.at[pl.ds(ebase + b * K, K)], si_v)
        pltpu.sync_copy(dstx_hbm.at[pl.ds(ebase + b * K, K)], di_v)
        pltpu.sync_copy(wall_hbm.at[pl.ds(ebase + b * K, K)], w_v)
        pltpu.async_copy(xp_hbm.at[si_v], b_v, sem)

    def _triple(i, carry):
        co = i * 3
        for b in range(3):
            si_v, di_v, w_v, b_v, sem = bufs[b]
            ci = co + b

            @pl.when(ci < cnt)
            def _():
                pltpu.make_async_copy(xp_hbm.at[si_v], b_v, sem).wait()

                def _scale_group(gi, carry2):
                    o = gi * L
                    wg = w_v[pl.ds(o, L)]
                    for k in range(L):
                        wb = jnp.full((L,), wg[k], jnp.float32)
                        for j in range(C // L):
                            b_v[o + k, pl.ds(j * L, L)] = (
                                b_v[o + k, pl.ds(j * L, L)] * wb)
                    return carry2

                lax.fori_loop(0, K // L, _scale_group, 0)
                pltpu.sync_copy(b_v, acc_sh.at[di_v], add=True)

                nci = ci + 3

                @pl.when(nci < cnt)
                def _():
                    nb = ebase + nci * K
                    pltpu.sync_copy(srcx_hbm.at[pl.ds(nb, K)], si_v)
                    pltpu.sync_copy(dstx_hbm.at[pl.ds(nb, K)], di_v)
                    pltpu.sync_copy(wall_hbm.at[pl.ds(nb, K)], w_v)
                    pltpu.async_copy(xp_hbm.at[si_v], b_v, sem)

        return carry

    lax.fori_loop(0, (cnt + 2) // 3, _triple, 0)

    plsc.subcore_barrier()

    @pl.when(cid == 0)
    def _():
        for r in range(RPT // KA):
            ro = rbase + r * KA
            pltpu.sync_copy(acc_sh.at[pl.ds(ro, KA)],
                            out0_hbm.at[pl.ds(ro, KA)])

    @pl.when(cid == 1)
    def _():
        for r in range(RPT // KA):
            ro = rbase + r * KA
            pltpu.sync_copy(acc_sh.at[pl.ds(ro, KA)],
                            out1_hbm.at[pl.ds(ro, KA)])


def _norm_body(acc0_ref, acc1_ref, ws_ref, xp_ref, asrc_ref, adst_ref,
               gs_ref, gd_ref, bias_ref, out_ref):
    g = jnp.maximum(gs_ref[0, 0] + gd_ref[0, 0], 0.0)
    a = asrc_ref[...] + adst_ref[...]
    a = jnp.where(a > 0, a, NEG_SLOPE * a)
    w = jnp.exp(a - g)
    s = lax.dot_general(ws_ref[...], jnp.ones((NW, 1), jnp.float32),
                        (((0,), (0,)), ((), ())),
                        preferred_element_type=jnp.float32)
    y = (acc0_ref[...] + acc1_ref[...]) + w * xp_ref[...]
    out_ref[...] = y / (s + w + 1e-16) + bias_ref[...]


def _normalize(acc0, acc1, ws, xp, asrc, adst, gs, gd, bias):
    BN = 1024
    return pl.pallas_call(
        _norm_body,
        grid=(NP // BN,),
        in_specs=[
            pl.BlockSpec((BN, C), lambda i: (i, 0)),
            pl.BlockSpec((BN, C), lambda i: (i, 0)),
            pl.BlockSpec((NW, BN), lambda i: (0, i)),
            pl.BlockSpec((BN, C), lambda i: (i, 0)),
            pl.BlockSpec((BN, 1), lambda i: (i, 0)),
            pl.BlockSpec((BN, 1), lambda i: (i, 0)),
            pl.BlockSpec((1, 1), lambda i: (0, 0),
                         memory_space=pltpu.MemorySpace.SMEM),
            pl.BlockSpec((1, 1), lambda i: (0, 0),
                         memory_space=pltpu.MemorySpace.SMEM),
            pl.BlockSpec((1, C), lambda i: (0, 0)),
        ],
        out_specs=pl.BlockSpec((BN, C), lambda i: (i, 0)),
        out_shape=jax.ShapeDtypeStruct((N, C), jnp.float32),
    )(acc0, acc1, ws, xp, asrc, adst, gs, gd, bias)


def kernel(x, edge_index, W_src, att_src, att_dst, bias):
    att_s = att_src.reshape(1, C)
    att_d = att_dst.reshape(1, C)

    asrc, adst, gs, gd = _alphas(x, W_src, att_s, att_d)

    g = jnp.maximum(gs[0, 0] + gd[0, 0], 0.0)
    g16 = jnp.broadcast_to(g.reshape(1), (L,)).astype(jnp.float32)

    zpad = jnp.zeros((ETOT - E,), jnp.int32)
    srcx = jnp.concatenate([edge_index[0], zpad])
    dstx = jnp.concatenate([edge_index[1], zpad])

    wall, ws = _weights_kernel(asrc.reshape(N), adst.reshape(N), g16,
                               srcx, dstx)
    xp = _project(x, W_src)
    acc0, acc1 = _rows_kernel(srcx, dstx, wall, xp)

    out = _normalize(acc0, acc1, ws, xp, asrc, adst, gs, gd,
                     bias.reshape(1, C))
    return out
